# baseline (speedup 1.0000x reference)
import functools

import jax
import jax.numpy as jnp
from jax import lax
from jax.experimental import pallas as pl
from jax.experimental.pallas import tpu as pltpu
from jax.experimental.pallas import tpu_sc as plsc

BATCH = 16384
D = 16
NC = 2
NS = 16
NW = NC * NS
BW = BATCH // NW
CHUNK = 128
NCHUNK = BW // CHUNK

_mesh = plsc.VectorSubcoreMesh(core_axis_name="c", subcore_axis_name="s")


@functools.partial(
    pl.kernel,
    mesh=_mesh,
    out_type=jax.ShapeDtypeStruct((BATCH,), jnp.float32),
    scratch_types=[
        pltpu.VMEM((NCHUNK, CHUNK), jnp.int32),
        pltpu.VMEM((NCHUNK, CHUNK), jnp.int32),
        pltpu.VMEM((BW, D), jnp.float32),
        pltpu.VMEM((BW, D), jnp.float32),
        pltpu.VMEM((BW,), jnp.float32),
        pltpu.SemaphoreType.DMA,
    ],
    compiler_params=pltpu.CompilerParams(
        needs_layout_passes=False, use_tc_tiling_on_sc=False),
)
def _mf_sc(xu_hbm, xv_hbm, uemb_hbm, vemb_hbm, out_hbm,
           xu_v, xv_v, urows, vrows, out_v, sem):
    wid = lax.axis_index("s") * NC + lax.axis_index("c")
    base = wid * BW

    pltpu.sync_copy(xu_hbm.at[pl.ds(wid * NCHUNK, NCHUNK), :], xu_v)
    pltpu.sync_copy(xv_hbm.at[pl.ds(wid * NCHUNK, NCHUNK), :], xv_v)

    copies = []
    for j in range(NCHUNK):
        copies.append(pltpu.async_copy(
            uemb_hbm.at[xu_v.at[j]], urows.at[pl.ds(j * CHUNK, CHUNK), :], sem))
        copies.append(pltpu.async_copy(
            vemb_hbm.at[xv_v.at[j]], vrows.at[pl.ds(j * CHUNK, CHUNK), :], sem))
    for c in copies:
        c.wait()

    lane_ids = lax.iota(jnp.int32, 16)
    def blk_body(b, carry):
        rows_idx = b * 16 + lane_ids
        acc = jnp.zeros((16,), jnp.float32)
        for d in range(D):
            cols = jnp.full((16,), d, jnp.int32)
            acc = acc + (plsc.load_gather(urows, [rows_idx, cols])
                         * plsc.load_gather(vrows, [rows_idx, cols]))
        out_v[pl.ds(b * 16, 16)] = acc
        return carry
    lax.fori_loop(0, BW // 16, blk_body, 0, unroll=2)

    pltpu.sync_copy(out_v, out_hbm.at[pl.ds(base, BW)])


def kernel(X, user_emb, item_emb):
    xu = X[:, 0].reshape(NW * NCHUNK, CHUNK)
    xv = X[:, 1].reshape(NW * NCHUNK, CHUNK)
    out = _mf_sc(xu, xv, user_emb, item_emb)
    return out.reshape(BATCH, 1)
